# Initial kernel scaffold; baseline (speedup 1.0000x reference)
import jax
import jax.numpy as jnp
from jax.experimental import pallas as pl


def kernel(x, weight):
    raise NotImplementedError("write your pallas kernel here")


# speedup vs baseline: 13.1824x; 13.1824x over previous
import functools

import jax
import jax.numpy as jnp
from jax import lax
from jax.experimental import pallas as pl
from jax.experimental.pallas import tpu as pltpu
from jax.experimental.pallas import tpu_sc as plsc

B = 4096
L = 200
D = 128
V = 100000

NC = 2
NS = 16
NW = NC * NS
RPW = B // NW
XPW = RPW * L

_INV_V = 1.0 / V


def _mod_v(v):
    q = (v.astype(jnp.float32) * _INV_V).astype(jnp.int32)
    r = v - q * V
    r = jnp.where(r < 0, r + V, r)
    r = jnp.where(r >= V, r - V, r)
    return r


def _sc_body(x_hbm, w_hbm, out_hbm, idxf, buf, outb, sem0, sem1):
    c = lax.axis_index("c")
    s = lax.axis_index("s")
    wid = s * NC + c
    xbase = wid * XPW
    obase = wid * RPW

    pltpu.sync_copy(x_hbm.at[pl.ds(xbase, XPW)], idxf)

    def mod_body(j, carry):
        base = j * 64
        for u in range(4):
            o = base + u * 16
            idxf[pl.ds(o, 16)] = _mod_v(idxf[pl.ds(o, 16)])
        return carry

    lax.fori_loop(0, XPW // 64, mod_body, 0)

    def issue(row, pbuf):
        sem = sem0 if pbuf == 0 else sem1
        ia = idxf.at[pl.ds(row * L, 128)]
        ib = idxf.at[pl.ds(row * L + 128, 72)]
        pltpu.make_async_copy(w_hbm.at[ia], buf.at[pbuf, pl.ds(0, 128)], sem).start()
        pltpu.make_async_copy(w_hbm.at[ib], buf.at[pbuf, pl.ds(128, 72)], sem).start()

    def wait(pbuf):
        sem = sem0 if pbuf == 0 else sem1
        pltpu.make_async_copy(
            w_hbm.at[pl.ds(0, 128)], buf.at[pbuf, pl.ds(0, 128)], sem).wait()
        pltpu.make_async_copy(
            w_hbm.at[pl.ds(0, 72)], buf.at[pbuf, pl.ds(128, 72)], sem).wait()

    def accum(pbuf, row):
        def rbody(t, accs):
            new = list(accs)
            for u in range(4):
                r = t * 4 + u
                for cc in range(8):
                    new[cc] = new[cc] + buf[pbuf, r, pl.ds(cc * 16, 16)]
            return tuple(new)

        z = jnp.zeros((16,), jnp.float32)
        accs = lax.fori_loop(0, L // 4, rbody, (z,) * 8)
        for cc in range(8):
            outb[row, pl.ds(cc * 16, 16)] = accs[cc]

    issue(0, 0)
    issue(1, 1)

    def gbody(g, carry):
        b0 = g * 2
        wait(0)
        accum(0, b0)

        @pl.when(g < RPW // 2 - 1)
        def _():
            issue(b0 + 2, 0)

        wait(1)
        accum(1, b0 + 1)

        @pl.when(g < RPW // 2 - 1)
        def _():
            issue(b0 + 3, 1)

        return carry

    lax.fori_loop(0, RPW // 2, gbody, 0)

    pltpu.sync_copy(outb, out_hbm.at[pl.ds(obase, RPW)])


_mesh = plsc.VectorSubcoreMesh(core_axis_name="c", subcore_axis_name="s")

_sc_kernel = pl.kernel(
    _sc_body,
    out_type=jax.ShapeDtypeStruct((B, D), jnp.float32),
    mesh=_mesh,
    scratch_types=[
        pltpu.VMEM((XPW,), jnp.int32),
        pltpu.VMEM((2, L, D), jnp.float32),
        pltpu.VMEM((RPW, D), jnp.float32),
        pltpu.SemaphoreType.DMA,
        pltpu.SemaphoreType.DMA,
    ],
)


@jax.jit
def kernel(x, weight):
    return _sc_kernel(x.reshape(-1), weight)
